# baseline (speedup 1.0000x reference)
import functools

import jax
import jax.numpy as jnp
from jax import lax
from jax.experimental import pallas as pl
from jax.experimental.pallas import tpu as pltpu
from jax.experimental.pallas import tpu_sc as plsc

_R = 800


def kernel(x, pos_emb):
    B, L = x.shape
    V, D = pos_emb.shape
    N = B * L
    NW = 32
    C = N // (NW * _R)
    G = _R // 16
    P = _R // 2
    LP = L // 2

    xf = x.reshape(N)
    bodyblk = pos_emb[1:L + 1].reshape(LP, 2 * D)
    row0 = pos_emb[0]

    mesh = plsc.VectorSubcoreMesh(core_axis_name="c", subcore_axis_name="s")

    @functools.partial(
        pl.kernel,
        mesh=mesh,
        out_type=jax.ShapeDtypeStruct((N // 2, 2 * D), jnp.float32),
        scratch_types=[
            pltpu.VMEM((LP, 2 * D), jnp.float32),
            pltpu.VMEM((D,), jnp.float32),
            pltpu.VMEM((_R,), jnp.int32),
            pltpu.VMEM((_R,), jnp.int32),
            pltpu.VMEM((P, 2 * D), jnp.float32),
            pltpu.VMEM((P, 2 * D), jnp.float32),
            pltpu.VMEM((_R,), jnp.int32),
            pltpu.VMEM((_R,), jnp.int32),
            pltpu.SemaphoreType.DMA,
            pltpu.SemaphoreType.DMA,
            pltpu.SemaphoreType.DMA,
            pltpu.SemaphoreType.DMA,
        ],
    )
    def sc_kern(xf_hbm, body_hbm, row0_hbm, out_hbm,
                tmpl_v, row0_v, x0_v, x1_v, rows0_v, rows1_v,
                bits0_v, bits1_v, sem0, sem1, xsem0, xsem1):
        wid = lax.axis_index("s") * 2 + lax.axis_index("c")
        xbufs = (x0_v, x1_v)
        rbufs = (rows0_v, rows1_v)
        bbufs = (bits0_v, bits1_v)
        sems = (sem0, sem1)
        xsems = (xsem0, xsem1)

        pltpu.sync_copy(body_hbm, tmpl_v)
        pltpu.sync_copy(row0_hbm, row0_v)
        for rv in rbufs:
            for k in range(P // LP):
                pltpu.sync_copy(body_hbm, rv.at[pl.ds(k * LP, LP)])

        def fix_rows(rv, g, bits, target_tmpl):
            def fix(i, carry):
                line = g * 16 + i
                p = lax.div(line, 2)
                h = lax.rem(line, 2)

                @pl.when(lax.shift_right_logical(bits, i) & 1 != 0)
                def _():
                    for c in range(D // 16):
                        sl = pl.ds(h * D + c * 16, 16)
                        if target_tmpl:
                            rv[p, sl] = tmpl_v[lax.rem(p, LP), sl]
                        else:
                            rv[p, sl] = row0_v[pl.ds(c * 16, 16)]
                return carry

            lax.fori_loop(0, 16, fix, 0)

        def group_bits(x_v, g):
            lanes = lax.iota(jnp.int32, 16)
            xv = x_v[pl.ds(g * 16, 16)]
            bitv = jnp.where(xv == 0, lax.shift_left(1, lanes), 0)
            bits = bitv[0]
            for i in range(1, 16):
                bits = bits | bitv[i]
            return bits

        def mask_patch(rv, x_v, bb_v):
            def per_group(g, carry):
                bits = group_bits(x_v, g)
                bb_v[pl.ds(g * 16, 16)] = lax.broadcast(bits, (16,))

                @pl.when(bits != 0)
                def _():
                    fix_rows(rv, g, bits, False)
                return carry

            lax.fori_loop(0, G, per_group, 0)

        def restore_patch(rv, bb_v):
            def per_group(g, carry):
                bits = bb_v[pl.ds(g * 16, 16)][0]

                @pl.when(bits != 0)
                def _():
                    fix_rows(rv, g, bits, True)
                return carry

            lax.fori_loop(0, G, per_group, 0)

        def has_zero(x_v):
            acc = x_v[pl.ds(0, 16)]
            for g in range(1, G):
                acc = jnp.minimum(acc, x_v[pl.ds(g * 16, 16)])
            m = acc[0]
            for i in range(1, 16):
                m = jnp.minimum(m, acc[i])
            return m == 0

        def run_chunk(c, b, not_first, dirty_prev):
            base = (wid * C + c) * _R
            pbase = (wid * C + c) * P

            @pl.when(not_first)
            def _():
                pltpu.make_async_copy(
                    rbufs[b], out_hbm.at[pl.ds(0, P)], sems[b]).wait()

            @pl.when(jnp.logical_and(not_first, dirty_prev))
            def _():
                restore_patch(rbufs[b], bbufs[b])

            pltpu.make_async_copy(
                xf_hbm.at[pl.ds(0, _R)], xbufs[b], xsems[b]).wait()
            dirty = has_zero(xbufs[b])

            @pl.when(dirty)
            def _():
                mask_patch(rbufs[b], xbufs[b], bbufs[b])

            pltpu.async_copy(
                rbufs[b], out_hbm.at[pl.ds(pbase, P)], sems[b])

            @pl.when(c + 2 < C)
            def _():
                pltpu.async_copy(
                    xf_hbm.at[pl.ds(base + 2 * _R, _R)], xbufs[b], xsems[b])
            return dirty

        for b in range(2):
            pltpu.async_copy(
                xf_hbm.at[pl.ds((wid * C + b) * _R, _R)], xbufs[b], xsems[b])

        def pair(g, carry):
            d0, d1 = carry
            d0 = run_chunk(2 * g, 0, g > 0, d0)
            d1 = run_chunk(2 * g + 1, 1, g > 0, d1)
            return d0, d1

        lax.fori_loop(0, C // 2, pair, (jnp.bool_(False), jnp.bool_(False)))
        for b in range(2):
            pltpu.make_async_copy(
                rbufs[b], out_hbm.at[pl.ds(0, P)], sems[b]).wait()

    out = sc_kern(xf, bodyblk, row0)
    return out.reshape(B, L, D)


# speedup vs baseline: 1.6658x; 1.6658x over previous
import functools

import jax
import jax.numpy as jnp
from jax import lax
from jax.experimental import pallas as pl
from jax.experimental.pallas import tpu as pltpu
from jax.experimental.pallas import tpu_sc as plsc

_R = 400


def kernel(x, pos_emb):
    B, L = x.shape
    V, D = pos_emb.shape
    N = B * L
    NW = 32
    C = N // (NW * _R)
    G = _R // 16

    xf = x.reshape(N)
    bodyblk = pos_emb[1:L + 1]
    row0 = pos_emb[0]

    mesh = plsc.VectorSubcoreMesh(core_axis_name="c", subcore_axis_name="s")

    @functools.partial(
        pl.kernel,
        mesh=mesh,
        out_type=jax.ShapeDtypeStruct((N, D), jnp.float32),
        compiler_params=pltpu.CompilerParams(use_tc_tiling_on_sc=True),
        scratch_types=[
            pltpu.VMEM((L, D), jnp.float32),
            pltpu.VMEM((D,), jnp.float32),
            pltpu.VMEM((_R,), jnp.int32),
            pltpu.VMEM((_R,), jnp.int32),
            pltpu.VMEM((_R, D), jnp.float32),
            pltpu.VMEM((_R, D), jnp.float32),
            pltpu.VMEM((_R,), jnp.int32),
            pltpu.VMEM((_R,), jnp.int32),
            pltpu.SemaphoreType.DMA,
            pltpu.SemaphoreType.DMA,
            pltpu.SemaphoreType.DMA,
            pltpu.SemaphoreType.DMA,
        ],
    )
    def sc_kern(xf_hbm, body_hbm, row0_hbm, out_hbm,
                tmpl_v, row0_v, x0_v, x1_v, rows0_v, rows1_v,
                bits0_v, bits1_v, sem0, sem1, xsem0, xsem1):
        wid = lax.axis_index("s") * 2 + lax.axis_index("c")
        xbufs = (x0_v, x1_v)
        rbufs = (rows0_v, rows1_v)
        bbufs = (bits0_v, bits1_v)
        sems = (sem0, sem1)
        xsems = (xsem0, xsem1)

        pltpu.sync_copy(body_hbm, tmpl_v)
        pltpu.sync_copy(row0_hbm, row0_v)
        for rv in rbufs:
            for k in range(_R // L):
                pltpu.sync_copy(body_hbm, rv.at[pl.ds(k * L, L)])

        def fix_rows(rv, g, bits, target_tmpl):
            def fix(i, carry):
                line = g * 16 + i

                @pl.when(lax.shift_right_logical(bits, i) & 1 != 0)
                def _():
                    for c in range(D // 16):
                        sl = pl.ds(c * 16, 16)
                        if target_tmpl:
                            rv[line, sl] = tmpl_v[lax.rem(line, L), sl]
                        else:
                            rv[line, sl] = row0_v[sl]
                return carry

            lax.fori_loop(0, 16, fix, 0)

        def mask_patch(rv, x_v, bb_v):
            lanes = lax.iota(jnp.int32, 16)
            for g in range(G):
                xv = x_v[pl.ds(g * 16, 16)]
                bitv = jnp.where(xv == 0, lax.shift_left(1, lanes), 0)
                bits = bitv[0]
                for i in range(1, 16):
                    bits = bits | bitv[i]
                bb_v[pl.ds(g * 16, 16)] = lax.broadcast(bits, (16,))

                @pl.when(bits != 0)
                def _():
                    fix_rows(rv, g, bits, False)

        def restore_patch(rv, bb_v):
            for g in range(G):
                bits = bb_v[pl.ds(g * 16, 16)][0]

                @pl.when(bits != 0)
                def _():
                    fix_rows(rv, g, bits, True)

        def has_zero(x_v):
            acc = x_v[pl.ds(0, 16)]
            for g in range(1, G):
                acc = jnp.minimum(acc, x_v[pl.ds(g * 16, 16)])
            m = acc[0]
            for i in range(1, 16):
                m = jnp.minimum(m, acc[i])
            return m == 0

        def run_chunk(c, b, not_first, dirty_prev):
            base = (wid * C + c) * _R

            @pl.when(not_first)
            def _():
                pltpu.make_async_copy(
                    rbufs[b], out_hbm.at[pl.ds(0, _R)], sems[b]).wait()

            @pl.when(jnp.logical_and(not_first, dirty_prev))
            def _():
                restore_patch(rbufs[b], bbufs[b])

            pltpu.make_async_copy(
                xf_hbm.at[pl.ds(0, _R)], xbufs[b], xsems[b]).wait()
            dirty = has_zero(xbufs[b])

            @pl.when(dirty)
            def _():
                mask_patch(rbufs[b], xbufs[b], bbufs[b])

            pltpu.async_copy(
                rbufs[b], out_hbm.at[pl.ds(base, _R)], sems[b])

            @pl.when(c + 2 < C)
            def _():
                pltpu.async_copy(
                    xf_hbm.at[pl.ds(base + 2 * _R, _R)], xbufs[b], xsems[b])
            return dirty

        for b in range(2):
            pltpu.async_copy(
                xf_hbm.at[pl.ds((wid * C + b) * _R, _R)], xbufs[b], xsems[b])

        def pair(g, carry):
            d0, d1 = carry
            d0 = run_chunk(2 * g, 0, g > 0, d0)
            d1 = run_chunk(2 * g + 1, 1, g > 0, d1)
            return d0, d1

        lax.fori_loop(0, C // 2, pair, (jnp.bool_(False), jnp.bool_(False)))
        for b in range(2):
            pltpu.make_async_copy(
                rbufs[b], out_hbm.at[pl.ds(0, _R)], sems[b]).wait()

    out = sc_kern(xf, bodyblk, row0)
    return out.reshape(B, L, D)
